# baseline (speedup 1.0000x reference)
import functools

import jax
import jax.numpy as jnp
from jax.experimental import pallas as pl
from jax.experimental.pallas import tpu as pltpu


def _compute_block(e_blk, w1, b1, w2):
    S, _, D = e_blk.shape
    slices = []
    logits = []
    for s in range(S):
        e = e_blk[s]
        eb = e.astype(jnp.bfloat16)
        h = jnp.maximum(
            jnp.dot(eb, w1, preferred_element_type=jnp.float32) + b1, 0.0)
        lt = jax.lax.dot_general(
            w2, h.astype(jnp.bfloat16), (((1,), (1,)), ((), ())),
            preferred_element_type=jnp.float32)
        slices.append(e)
        logits.append(lt)

    lg = jnp.concatenate(logits, axis=0)
    m = jnp.max(lg, axis=0, keepdims=True)
    ex = jnp.exp(lg - m)
    pn = ex / jnp.sum(ex, axis=0, keepdims=True)

    acc = None
    for s in range(S):
        sel = (jax.lax.broadcasted_iota(jnp.int32, (S, D), 0) == s)
        p_rep = jax.lax.dot_general(
            pn, sel.astype(jnp.float32), (((0,), (0,)), ((), ())),
            preferred_element_type=jnp.float32)
        term = p_rep * slices[s]
        acc = term if acc is None else acc + term
    return acc


def _agg_kernel(e_hbm, w1_ref, b1_ref, w2_ref, o_ref, buf, sem):
    i = pl.program_id(0)
    n = pl.num_programs(0)
    bblk = o_ref.shape[0]

    def start_copy(block, slot):
        pltpu.make_async_copy(
            e_hbm.at[:, pl.ds(block * bblk, bblk), :],
            buf.at[slot],
            sem.at[slot],
        ).start()

    @pl.when(i == 0)
    def _():
        start_copy(0, 0)

    @pl.when(i + 1 < n)
    def _():
        start_copy(i + 1, jax.lax.rem(i + 1, 2))

    slot = jax.lax.rem(i, 2)
    pltpu.make_async_copy(
        e_hbm.at[:, pl.ds(i * bblk, bblk), :], buf.at[slot], sem.at[slot]
    ).wait()

    o_ref[...] = _compute_block(
        buf[slot], w1_ref[...], b1_ref[...], w2_ref[...])


@functools.partial(jax.jit, static_argnames=("interpret",))
def kernel(user_embeds_list, userIdx, W1, b1, W2, b2, interpret=False):
    del userIdx, b2
    S, B, D = user_embeds_list.shape
    H = W1.shape[1]
    bblk = min(B, 1024)

    return pl.pallas_call(
        _agg_kernel,
        grid=(B // bblk,),
        in_specs=[
            pl.BlockSpec(memory_space=pltpu.MemorySpace.HBM),
            pl.BlockSpec((D, H), lambda i: (0, 0)),
            pl.BlockSpec((1, H), lambda i: (0, 0)),
            pl.BlockSpec((1, H), lambda i: (0, 0)),
        ],
        out_specs=pl.BlockSpec((bblk, D), lambda i: (i, 0)),
        out_shape=jax.ShapeDtypeStruct((B, D), jnp.float32),
        scratch_shapes=[
            pltpu.VMEM((2, S, bblk, D), jnp.float32),
            pltpu.SemaphoreType.DMA((2,)),
        ],
        interpret=interpret,
    )(
        user_embeds_list.astype(jnp.float32),
        W1.astype(jnp.bfloat16),
        b1.reshape(1, H).astype(jnp.float32),
        W2.reshape(1, H).astype(jnp.bfloat16),
    )
